# baseline (speedup 1.0000x reference)
import jax
import jax.numpy as jnp
from jax.experimental import pallas as pl

D_EMB = 1000
ROWS = 4096
COLS = 20
BLK = 64


def _onehot_block(x_ref, o_ref):
    xb = x_ref[...]
    iota = jax.lax.broadcasted_iota(jnp.int32, (BLK, COLS, D_EMB), 2)
    o_ref[...] = jnp.where(xb[:, :, None] == iota, 5.0, 0.0).astype(jnp.float32)


def kernel(x):
    grid = (ROWS // BLK,)
    return pl.pallas_call(
        _onehot_block,
        grid=grid,
        in_specs=[pl.BlockSpec((BLK, COLS), lambda i: (i, 0))],
        out_specs=pl.BlockSpec((BLK, COLS, D_EMB), lambda i: (i, 0, 0)),
        out_shape=jax.ShapeDtypeStruct((ROWS, COLS, D_EMB), jnp.float32),
    )(x)
